# baseline (speedup 1.0000x reference)
import functools

import jax
import jax.numpy as jnp
from jax import lax
from jax.experimental import pallas as pl
from jax.experimental.pallas import tpu as pltpu
from jax.experimental.pallas import tpu_sc as plsc


def _gather_rows_sc(table, idx):
    n_rows, d = table.shape
    s = idx.shape[0]
    info = plsc.get_sparse_core_info()
    nc, ns = info.num_cores, info.num_subcores
    nw = nc * ns
    assert s % nw == 0 and d % info.num_lanes == 0
    b_per_w = s // nw
    mesh = plsc.VectorSubcoreMesh(core_axis_name="c", subcore_axis_name="s")

    @functools.partial(
        pl.kernel,
        out_type=jax.ShapeDtypeStruct((s, d), table.dtype),
        mesh=mesh,
        scratch_types=[
            pltpu.VMEM((b_per_w,), jnp.int32),
            pltpu.VMEM((b_per_w, d), table.dtype),
            pltpu.SemaphoreType.DMA,
        ],
    )
    def gather_kernel(table_hbm, idx_hbm, out_hbm, idx_v, rows_v, sem):
        wid = lax.axis_index("s") * nc + lax.axis_index("c")
        base = wid * b_per_w
        pltpu.sync_copy(idx_hbm.at[pl.ds(base, b_per_w)], idx_v)
        pltpu.async_copy(table_hbm.at[idx_v], rows_v, sem).wait()
        pltpu.sync_copy(rows_v, out_hbm.at[pl.ds(base, b_per_w)])

    return gather_kernel(table, idx)


def _work_list(sei, n_experts, blk):
    s = sei.shape[0]
    nb = s // blk
    nw = nb + n_experts - 1
    blocks = sei.reshape(nb, blk)
    first = blocks[:, 0].astype(jnp.int32)
    last = blocks[:, -1].astype(jnp.int32)
    counts = last - first + 1
    cum = jnp.cumsum(counts)
    total = cum[-1]
    cumstart = cum - counts
    w_ids = jnp.arange(nw, dtype=jnp.int32)
    blk_of_w = jnp.minimum(
        jnp.searchsorted(cum, w_ids, side="right").astype(jnp.int32), nb - 1
    )
    expert_w = first[blk_of_w] + (w_ids - cumstart[blk_of_w])
    valid_w = w_ids < total
    expert_w = jnp.where(valid_w, expert_w, last[nb - 1]).astype(jnp.int32)
    first_w = ((w_ids == cumstart[blk_of_w]) & valid_w).astype(jnp.int32)
    last_w = ((w_ids == cum[blk_of_w] - 1) & valid_w).astype(jnp.int32)
    return blk_of_w, expert_w, first_w, last_w, nw


_NBUF = 4


def _fetch_schedule(exp_w, nw):
    prev = jnp.concatenate([exp_w[:1] - 1, exp_w[:-1]])
    run_start = (exp_w != prev).astype(jnp.int32)
    rid = jnp.cumsum(run_start) - 1
    nrun = rid[-1] + 1
    run_expert = jnp.zeros((nw,), jnp.int32).at[rid].set(exp_w)
    use_slot = (rid % _NBUF).astype(jnp.int32)
    jw = rid + _NBUF - 1
    iss_val = (run_start == 1) & (jw < nrun)
    iss_exp = run_expert[jnp.minimum(jw, nw - 1)]
    iss_slot = (jw % _NBUF).astype(jnp.int32)
    return (run_start, use_slot, iss_val.astype(jnp.int32), iss_exp,
            iss_slot, run_expert,
            jnp.full((1,), nrun, jnp.int32))


def _make_fused_body(blk, kk, d_out):
    tok = blk // kk

    def body(blk_r, exp_r, fst_r, lst_r, wait_r, slot_r, iv_r, ie_r, is_r,
             rexp_r, nrun_r, x_ref, w_hbm, sg_ref, out_ref,
             acc_ref, wbufs, sems):
        w = pl.program_id(0)

        @pl.when(w == 0)
        def _():
            for i in range(_NBUF - 1):
                @pl.when(i < nrun_r[0])
                def _():
                    pltpu.make_async_copy(
                        w_hbm.at[rexp_r[i]], wbufs.at[i], sems.at[i]).start()

        @pl.when(iv_r[w] == 1)
        def _():
            pltpu.make_async_copy(
                w_hbm.at[ie_r[w]], wbufs.at[is_r[w]], sems.at[is_r[w]]
            ).start()

        slot = slot_r[w]
        e = exp_r[w]

        @pl.when(wait_r[w] == 1)
        def _():
            pltpu.make_async_copy(
                w_hbm.at[e], wbufs.at[slot], sems.at[slot]).wait()

        sg = sg_ref[...]
        mask = (sg[:, 0:1] == e.astype(jnp.float32)).astype(jnp.float32)
        xm = x_ref[...] * (mask * sg[:, 1:2])
        contrib = lax.dot_general(
            xm, wbufs[slot], (((1,), (1,)), ((), ())),
            preferred_element_type=jnp.float32,
        )

        @pl.when(fst_r[w] == 1)
        def _():
            acc_ref[...] = contrib

        @pl.when(fst_r[w] == 0)
        def _():
            acc_ref[...] += contrib

        @pl.when(lst_r[w] == 1)
        def _():
            rows = lax.broadcasted_iota(jnp.int32, (tok, blk), 0)
            cols = lax.broadcasted_iota(jnp.int32, (tok, blk), 1)
            pair = (cols // kk == rows).astype(jnp.float32)
            out_ref[...] = lax.dot_general(
                pair, acc_ref[...], (((1,), (0,)), ((), ())),
                preferred_element_type=jnp.float32,
            )

    return body


def _grouped_matmul_combine(xg, weight, sei, gates, blk=512, interpret=False):
    s, d_in = xg.shape
    n_experts, d_out, _ = weight.shape
    t, kk = gates.shape
    assert blk % kk == 0 and s % blk == 0
    blk_w, exp_w, fst_w, lst_w, nw = _work_list(sei, n_experts, blk)
    sched = _fetch_schedule(exp_w, nw)
    sg = jnp.stack(
        [sei.astype(jnp.float32), gates.reshape(s).astype(jnp.float32)],
        axis=1)
    tok = blk // kk
    grid_spec = pltpu.PrefetchScalarGridSpec(
        num_scalar_prefetch=11,
        grid=(nw,),
        in_specs=[
            pl.BlockSpec((blk, d_in), lambda w, *r: (r[0][w], 0)),
            pl.BlockSpec(memory_space=pl.ANY),
            pl.BlockSpec((blk, 2), lambda w, *r: (r[0][w], 0)),
        ],
        out_specs=pl.BlockSpec((tok, d_out), lambda w, *r: (r[0][w], 0)),
        scratch_shapes=[
            pltpu.VMEM((blk, d_out), jnp.float32),
            pltpu.VMEM((_NBUF, d_out, d_in), jnp.float32),
            pltpu.SemaphoreType.DMA((_NBUF,)),
        ],
    )
    return pl.pallas_call(
        _make_fused_body(blk, kk, d_out),
        grid_spec=grid_spec,
        out_shape=jax.ShapeDtypeStruct((t, d_out), jnp.float32),
        compiler_params=pltpu.CompilerParams(
            dimension_semantics=("arbitrary",)),
        interpret=interpret,
    )(blk_w, exp_w, fst_w, lst_w, *sched, xg, weight, sg)


def kernel(inputs, weight, k, sorted_expert_idxs, sorted_scattered_idxs,
           expert_offsets, gates):
    del k, expert_offsets
    kk = gates.shape[1]
    token_idxs = (sorted_scattered_idxs // kk).astype(jnp.int32)
    xg = _gather_rows_sc(inputs, token_idxs)
    return _grouped_matmul_combine(xg, weight, sorted_expert_idxs, gates)


# speedup vs baseline: 1.5290x; 1.5290x over previous
import functools

import jax
import jax.numpy as jnp
from jax import lax
from jax.experimental import pallas as pl
from jax.experimental.pallas import tpu as pltpu
from jax.experimental.pallas import tpu_sc as plsc


def _gather_rows_sc(table, idx):
    n_rows, d = table.shape
    s = idx.shape[0]
    info = plsc.get_sparse_core_info()
    nc, ns = info.num_cores, info.num_subcores
    nw = nc * ns
    assert s % nw == 0 and d % info.num_lanes == 0
    b_per_w = s // nw
    mesh = plsc.VectorSubcoreMesh(core_axis_name="c", subcore_axis_name="s")

    @functools.partial(
        pl.kernel,
        out_type=jax.ShapeDtypeStruct((s, d), table.dtype),
        mesh=mesh,
        scratch_types=[
            pltpu.VMEM((b_per_w,), jnp.int32),
            pltpu.VMEM((b_per_w, d), table.dtype),
            pltpu.SemaphoreType.DMA,
        ],
    )
    def gather_kernel(table_hbm, idx_hbm, out_hbm, idx_v, rows_v, sem):
        wid = lax.axis_index("s") * nc + lax.axis_index("c")
        base = wid * b_per_w
        pltpu.sync_copy(idx_hbm.at[pl.ds(base, b_per_w)], idx_v)
        pltpu.async_copy(table_hbm.at[idx_v], rows_v, sem).wait()
        pltpu.sync_copy(rows_v, out_hbm.at[pl.ds(base, b_per_w)])

    return gather_kernel(table, idx)


def _work_list(sei, n_experts, blk):
    s = sei.shape[0]
    nb = s // blk
    nw = nb + n_experts - 1
    blocks = sei.reshape(nb, blk)
    first = blocks[:, 0].astype(jnp.int32)
    last = blocks[:, -1].astype(jnp.int32)
    counts = last - first + 1
    cum = jnp.cumsum(counts)
    total = cum[-1]
    cumstart = cum - counts
    w_ids = jnp.arange(nw, dtype=jnp.int32)
    blk_of_w = jnp.minimum(
        jnp.sum((cum[None, :] <= w_ids[:, None]).astype(jnp.int32), axis=1),
        nb - 1,
    ).astype(jnp.int32)
    oh = (blk_of_w[:, None] == jnp.arange(nb, dtype=jnp.int32)[None, :])
    ohi = oh.astype(jnp.int32)
    first_sel = jnp.sum(ohi * first[None, :], axis=1)
    cumstart_sel = jnp.sum(ohi * cumstart[None, :], axis=1)
    cum_sel = jnp.sum(ohi * cum[None, :], axis=1)
    expert_w = first_sel + (w_ids - cumstart_sel)
    valid_w = w_ids < total
    expert_w = jnp.where(valid_w, expert_w, last[nb - 1]).astype(jnp.int32)
    first_w = ((w_ids == cumstart_sel) & valid_w).astype(jnp.int32)
    last_w = ((w_ids == cum_sel - 1) & valid_w).astype(jnp.int32)
    return blk_of_w, expert_w, first_w, last_w, nw


_NBUF = 4


def _fetch_schedule(exp_w, nw):
    prev = jnp.concatenate([exp_w[:1] - 1, exp_w[:-1]])
    run_start = (exp_w != prev).astype(jnp.int32)
    rid = jnp.cumsum(run_start) - 1
    nrun = rid[-1] + 1
    r_ids = jnp.arange(nw, dtype=jnp.int32)
    starts = (rid[None, :] == r_ids[:, None]) & (run_start[None, :] == 1)
    run_expert = jnp.sum(starts.astype(jnp.int32) * exp_w[None, :], axis=1)
    use_slot = (rid % _NBUF).astype(jnp.int32)
    jw = rid + _NBUF - 1
    iss_val = (run_start == 1) & (jw < nrun)
    jm = jnp.minimum(jw, nw - 1)
    iss_exp = jnp.sum(
        ((rid[None, :] == jm[:, None]) & (run_start[None, :] == 1)).astype(
            jnp.int32) * exp_w[None, :], axis=1)
    iss_slot = (jw % _NBUF).astype(jnp.int32)
    return (run_start, use_slot, iss_val.astype(jnp.int32), iss_exp,
            iss_slot, run_expert,
            jnp.full((1,), nrun, jnp.int32))


def _make_fused_body(blk, kk, d_out):
    tok = blk // kk

    def body(blk_r, exp_r, fst_r, lst_r, wait_r, slot_r, iv_r, ie_r, is_r,
             rexp_r, nrun_r, x_ref, w_hbm, sg_ref, out_ref,
             acc_ref, wbufs, sems):
        w = pl.program_id(0)

        @pl.when(w == 0)
        def _():
            for i in range(_NBUF - 1):
                @pl.when(i < nrun_r[0])
                def _():
                    pltpu.make_async_copy(
                        w_hbm.at[rexp_r[i]], wbufs.at[i], sems.at[i]).start()

        @pl.when(iv_r[w] == 1)
        def _():
            pltpu.make_async_copy(
                w_hbm.at[ie_r[w]], wbufs.at[is_r[w]], sems.at[is_r[w]]
            ).start()

        slot = slot_r[w]
        e = exp_r[w]

        @pl.when(wait_r[w] == 1)
        def _():
            pltpu.make_async_copy(
                w_hbm.at[e], wbufs.at[slot], sems.at[slot]).wait()

        sg = sg_ref[...]
        mask = (sg[:, 0:1] == e.astype(jnp.float32)).astype(jnp.float32)
        xm = x_ref[...] * (mask * sg[:, 1:2])
        contrib = lax.dot_general(
            xm, wbufs[slot], (((1,), (1,)), ((), ())),
            preferred_element_type=jnp.float32,
        )

        @pl.when(fst_r[w] == 1)
        def _():
            acc_ref[...] = contrib

        @pl.when(fst_r[w] == 0)
        def _():
            acc_ref[...] += contrib

        @pl.when(lst_r[w] == 1)
        def _():
            rows = lax.broadcasted_iota(jnp.int32, (tok, blk), 0)
            cols = lax.broadcasted_iota(jnp.int32, (tok, blk), 1)
            pair = (cols // kk == rows).astype(jnp.float32)
            out_ref[...] = lax.dot_general(
                pair, acc_ref[...], (((1,), (0,)), ((), ())),
                preferred_element_type=jnp.float32,
            )

    return body


def _grouped_matmul_combine(xg, weight, sei, gates, blk=256, interpret=False):
    s, d_in = xg.shape
    n_experts, d_out, _ = weight.shape
    t, kk = gates.shape
    assert blk % kk == 0 and s % blk == 0
    blk_w, exp_w, fst_w, lst_w, nw = _work_list(sei, n_experts, blk)
    sched = _fetch_schedule(exp_w, nw)
    sg = jnp.stack(
        [sei.astype(jnp.float32), gates.reshape(s).astype(jnp.float32)],
        axis=1)
    tok = blk // kk
    grid_spec = pltpu.PrefetchScalarGridSpec(
        num_scalar_prefetch=11,
        grid=(nw,),
        in_specs=[
            pl.BlockSpec((blk, d_in), lambda w, *r: (r[0][w], 0)),
            pl.BlockSpec(memory_space=pl.ANY),
            pl.BlockSpec((blk, 2), lambda w, *r: (r[0][w], 0)),
        ],
        out_specs=pl.BlockSpec((tok, d_out), lambda w, *r: (r[0][w], 0)),
        scratch_shapes=[
            pltpu.VMEM((blk, d_out), jnp.float32),
            pltpu.VMEM((_NBUF, d_out, d_in), jnp.float32),
            pltpu.SemaphoreType.DMA((_NBUF,)),
        ],
    )
    return pl.pallas_call(
        _make_fused_body(blk, kk, d_out),
        grid_spec=grid_spec,
        out_shape=jax.ShapeDtypeStruct((t, d_out), jnp.float32),
        compiler_params=pltpu.CompilerParams(
            dimension_semantics=("arbitrary",)),
        interpret=interpret,
    )(blk_w, exp_w, fst_w, lst_w, *sched, xg, weight, sg)


def kernel(inputs, weight, k, sorted_expert_idxs, sorted_scattered_idxs,
           expert_offsets, gates):
    del k, expert_offsets
    kk = gates.shape[1]
    token_idxs = (sorted_scattered_idxs // kk).astype(jnp.int32)
    xg = _gather_rows_sc(inputs, token_idxs)
    return _grouped_matmul_combine(xg, weight, sorted_expert_idxs, gates)


# speedup vs baseline: 1.5489x; 1.0130x over previous
import functools

import jax
import jax.numpy as jnp
from jax import lax
from jax.experimental import pallas as pl
from jax.experimental.pallas import tpu as pltpu
from jax.experimental.pallas import tpu_sc as plsc


def _gather_rows_sc(table, raw_idx, kk):
    n_rows, d = table.shape
    s = raw_idx.shape[0]
    info = plsc.get_sparse_core_info()
    nc, ns = info.num_cores, info.num_subcores
    nw = nc * ns
    lanes = info.num_lanes
    assert s % nw == 0 and d % lanes == 0 and (s // nw) % lanes == 0
    b_per_w = s // nw
    mesh = plsc.VectorSubcoreMesh(core_axis_name="c", subcore_axis_name="s")

    @functools.partial(
        pl.kernel,
        out_type=jax.ShapeDtypeStruct((s, d), table.dtype),
        mesh=mesh,
        scratch_types=[
            pltpu.VMEM((b_per_w,), jnp.int32),
            pltpu.VMEM((b_per_w,), jnp.int32),
            pltpu.VMEM((b_per_w, d), table.dtype),
            pltpu.SemaphoreType.DMA,
        ],
    )
    def gather_kernel(table_hbm, idx_hbm, out_hbm, raw_v, idx_v, rows_v, sem):
        wid = lax.axis_index("s") * nc + lax.axis_index("c")
        base = wid * b_per_w
        pltpu.sync_copy(idx_hbm.at[pl.ds(base, b_per_w)], raw_v)
        shift = kk.bit_length() - 1
        for i in range(b_per_w // lanes):
            sl = pl.ds(i * lanes, lanes)
            idx_v[sl] = lax.shift_right_logical(raw_v[sl], shift)
        pltpu.async_copy(table_hbm.at[idx_v], rows_v, sem).wait()
        pltpu.sync_copy(rows_v, out_hbm.at[pl.ds(base, b_per_w)])

    return gather_kernel(table, raw_idx)


def _work_list(sei, n_experts, blk):
    s = sei.shape[0]
    nb = s // blk
    nw = nb + n_experts - 1
    blocks = sei.reshape(nb, blk)
    first = blocks[:, 0].astype(jnp.int32)
    last = blocks[:, -1].astype(jnp.int32)
    counts = last - first + 1
    b_ids = jnp.arange(nb, dtype=jnp.int32)
    cum = jnp.sum(
        (b_ids[None, :] <= b_ids[:, None]).astype(jnp.int32)
        * counts[None, :], axis=1)
    total = cum[-1]
    cumstart = cum - counts
    w_ids = jnp.arange(nw, dtype=jnp.int32)
    blk_of_w = jnp.minimum(
        jnp.sum((cum[None, :] <= w_ids[:, None]).astype(jnp.int32), axis=1),
        nb - 1,
    ).astype(jnp.int32)
    oh = (blk_of_w[:, None] == jnp.arange(nb, dtype=jnp.int32)[None, :])
    ohi = oh.astype(jnp.int32)
    first_sel = jnp.sum(ohi * first[None, :], axis=1)
    cumstart_sel = jnp.sum(ohi * cumstart[None, :], axis=1)
    cum_sel = jnp.sum(ohi * cum[None, :], axis=1)
    expert_w = first_sel + (w_ids - cumstart_sel)
    valid_w = w_ids < total
    expert_w = jnp.where(valid_w, expert_w, last[nb - 1]).astype(jnp.int32)
    first_w = ((w_ids == cumstart_sel) & valid_w).astype(jnp.int32)
    last_w = ((w_ids == cum_sel - 1) & valid_w).astype(jnp.int32)
    return blk_of_w, expert_w, first_w, last_w, nw


_NBUF = 4


def _fetch_schedule(exp_w, nw):
    prev = jnp.concatenate([exp_w[:1] - 1, exp_w[:-1]])
    run_start = (exp_w != prev).astype(jnp.int32)
    r_ids = jnp.arange(nw, dtype=jnp.int32)
    rid = jnp.sum(
        (r_ids[None, :] <= r_ids[:, None]).astype(jnp.int32)
        * run_start[None, :], axis=1) - 1
    nrun = rid[-1] + 1
    starts = (rid[None, :] == r_ids[:, None]) & (run_start[None, :] == 1)
    run_expert = jnp.sum(starts.astype(jnp.int32) * exp_w[None, :], axis=1)
    use_slot = (rid % _NBUF).astype(jnp.int32)
    jw = rid + _NBUF - 1
    iss_val = (run_start == 1) & (jw < nrun)
    jm = jnp.minimum(jw, nw - 1)
    iss_exp = jnp.sum(
        ((rid[None, :] == jm[:, None]) & (run_start[None, :] == 1)).astype(
            jnp.int32) * exp_w[None, :], axis=1)
    iss_slot = (jw % _NBUF).astype(jnp.int32)
    return (run_start, use_slot, iss_val.astype(jnp.int32), iss_exp,
            iss_slot, run_expert,
            jnp.full((1,), nrun, jnp.int32))


def _make_fused_body(blk, kk, d_out):
    tok = blk // kk

    def body(blk_r, exp_r, fst_r, lst_r, wait_r, slot_r, iv_r, ie_r, is_r,
             rexp_r, nrun_r, x_ref, w_hbm, sg_ref, out_ref,
             acc_ref, wbufs, sems):
        w = pl.program_id(0)

        @pl.when(w == 0)
        def _():
            for i in range(_NBUF - 1):
                @pl.when(i < nrun_r[0])
                def _():
                    pltpu.make_async_copy(
                        w_hbm.at[rexp_r[i]], wbufs.at[i], sems.at[i]).start()

        @pl.when(iv_r[w] == 1)
        def _():
            pltpu.make_async_copy(
                w_hbm.at[ie_r[w]], wbufs.at[is_r[w]], sems.at[is_r[w]]
            ).start()

        slot = slot_r[w]
        e = exp_r[w]

        @pl.when(wait_r[w] == 1)
        def _():
            pltpu.make_async_copy(
                w_hbm.at[e], wbufs.at[slot], sems.at[slot]).wait()

        sg = sg_ref[...]
        mask = (sg[:, 0:1] == e.astype(jnp.float32)).astype(jnp.float32)
        xm = x_ref[...] * (mask * sg[:, 1:2])
        contrib = lax.dot_general(
            xm, wbufs[slot], (((1,), (1,)), ((), ())),
            preferred_element_type=jnp.float32,
        )

        @pl.when(fst_r[w] == 1)
        def _():
            acc_ref[...] = contrib

        @pl.when(fst_r[w] == 0)
        def _():
            acc_ref[...] += contrib

        @pl.when(lst_r[w] == 1)
        def _():
            rows = lax.broadcasted_iota(jnp.int32, (tok, blk), 0)
            cols = lax.broadcasted_iota(jnp.int32, (tok, blk), 1)
            pair = (cols // kk == rows).astype(jnp.float32)
            out_ref[...] = lax.dot_general(
                pair, acc_ref[...], (((1,), (0,)), ((), ())),
                preferred_element_type=jnp.float32,
            )

    return body


def _grouped_matmul_combine(xg, weight, sei, gates, blk=256, interpret=False):
    s, d_in = xg.shape
    n_experts, d_out, _ = weight.shape
    t, kk = gates.shape
    assert blk % kk == 0 and s % blk == 0
    blk_w, exp_w, fst_w, lst_w, nw = _work_list(sei, n_experts, blk)
    sched = _fetch_schedule(exp_w, nw)
    sg = jnp.stack(
        [sei.astype(jnp.float32), gates.reshape(s).astype(jnp.float32)],
        axis=1)
    tok = blk // kk
    grid_spec = pltpu.PrefetchScalarGridSpec(
        num_scalar_prefetch=11,
        grid=(nw,),
        in_specs=[
            pl.BlockSpec((blk, d_in), lambda w, *r: (r[0][w], 0)),
            pl.BlockSpec(memory_space=pl.ANY),
            pl.BlockSpec((blk, 2), lambda w, *r: (r[0][w], 0)),
        ],
        out_specs=pl.BlockSpec((tok, d_out), lambda w, *r: (r[0][w], 0)),
        scratch_shapes=[
            pltpu.VMEM((blk, d_out), jnp.float32),
            pltpu.VMEM((_NBUF, d_out, d_in), jnp.float32),
            pltpu.SemaphoreType.DMA((_NBUF,)),
        ],
    )
    return pl.pallas_call(
        _make_fused_body(blk, kk, d_out),
        grid_spec=grid_spec,
        out_shape=jax.ShapeDtypeStruct((t, d_out), jnp.float32),
        compiler_params=pltpu.CompilerParams(
            dimension_semantics=("arbitrary",)),
        interpret=interpret,
    )(blk_w, exp_w, fst_w, lst_w, *sched, xg, weight, sg)


def kernel(inputs, weight, k, sorted_expert_idxs, sorted_scattered_idxs,
           expert_offsets, gates):
    del k, expert_offsets
    kk = gates.shape[1]
    ssi = sorted_scattered_idxs.astype(jnp.int32)
    if kk & (kk - 1) == 0:
        xg = _gather_rows_sc(inputs, ssi, kk)
    else:
        xg = _gather_rows_sc(inputs, ssi // kk, 1)
    return _grouped_matmul_combine(xg, weight, sorted_expert_idxs, gates)
